# Initial kernel scaffold; baseline (speedup 1.0000x reference)
import jax
import jax.numpy as jnp
from jax.experimental import pallas as pl


def kernel(x, edge_index, W1, b1, W2, b2):
    raise NotImplementedError("write your pallas kernel here")


# speedup vs baseline: 35.9501x; 35.9501x over previous
import functools

import jax
import jax.numpy as jnp
from jax import lax
from jax.experimental import pallas as pl
from jax.experimental.pallas import tpu as pltpu
from jax.experimental.pallas import tpu_sc as plsc

N_NODES = 10000
D_IN = 128
D_HID = 64
E_EDGES = 320000

NC = 2
NS = 16
NW = NC * NS
N_PAD = 10240
ROWS_PT = N_PAD // NS
EPW = E_EDGES // NW
CHUNK = 100
KCH = EPW // CHUNK

_MESH = plsc.VectorSubcoreMesh(
    core_axis_name="c", subcore_axis_name="s", num_cores=NC, num_subcores=NS)


def _wid():
    return lax.axis_index("s") * NC + lax.axis_index("c")


def _deg_body(dst_hbm, ones_hbm, zeros_hbm, out0_hbm, out1_hbm,
              idx_v, ones_v, acc_sh):
    c = lax.axis_index("c")
    s = lax.axis_index("s")
    pltpu.sync_copy(zeros_hbm.at[pl.ds(s * ROWS_PT, ROWS_PT)],
                    acc_sh.at[pl.ds(s * ROWS_PT, ROWS_PT)])
    pltpu.sync_copy(ones_hbm, ones_v)
    pltpu.sync_copy(dst_hbm.at[_wid()], idx_v)
    plsc.subcore_barrier()

    def body(g, carry):
        pltpu.sync_copy(ones_v, acc_sh.at[idx_v.at[g]], add=True)
        return carry

    lax.fori_loop(0, KCH, body, 0, unroll=False)
    plsc.subcore_barrier()

    @pl.when(c == 0)
    def _():
        pltpu.sync_copy(acc_sh.at[pl.ds(s * ROWS_PT, ROWS_PT)],
                        out0_hbm.at[pl.ds(s * ROWS_PT, ROWS_PT)])

    @pl.when(c == 1)
    def _():
        pltpu.sync_copy(acc_sh.at[pl.ds(s * ROWS_PT, ROWS_PT)],
                        out1_hbm.at[pl.ds(s * ROWS_PT, ROWS_PT)])


_SC_PARAMS = pltpu.CompilerParams(use_tc_tiling_on_sc=False)

_deg_kernel = pl.kernel(
    _deg_body,
    out_type=(jax.ShapeDtypeStruct((N_PAD, 16), jnp.float32),
              jax.ShapeDtypeStruct((N_PAD, 16), jnp.float32)),
    mesh=_MESH,
    scratch_types=[
        pltpu.VMEM((KCH, CHUNK), jnp.int32),
        pltpu.VMEM((CHUNK, 16), jnp.float32),
        pltpu.VMEM_SHARED((N_PAD, 16), jnp.float32),
    ],
    compiler_params=_SC_PARAMS,
)


def _agg_body(h_hbm, src_hbm, dst_hbm, zeros_hbm, out0_hbm, out1_hbm,
              src_v, dst_v, rows_a, rows_b, acc_sh, sem_a, sem_b):
    c = lax.axis_index("c")
    s = lax.axis_index("s")
    pltpu.sync_copy(zeros_hbm.at[pl.ds(s * ROWS_PT, ROWS_PT)],
                    acc_sh.at[pl.ds(s * ROWS_PT, ROWS_PT)])
    pltpu.sync_copy(src_hbm.at[_wid()], src_v)
    pltpu.sync_copy(dst_hbm.at[_wid()], dst_v)
    plsc.subcore_barrier()

    bufs = (rows_a, rows_b)
    sems = (sem_a, sem_b)

    pltpu.async_copy(h_hbm.at[src_v.at[0]], rows_a, sem_a)

    def outer(g, carry):
        for b in range(2):
            nxt = g + b + 1

            @pl.when(nxt < KCH)
            def _():
                pltpu.async_copy(h_hbm.at[src_v.at[nxt]],
                                 bufs[(b + 1) % 2], sems[(b + 1) % 2])

            pltpu.make_async_copy(h_hbm.at[src_v.at[g + b]],
                                  bufs[b], sems[b]).wait()
            pltpu.sync_copy(bufs[b], acc_sh.at[dst_v.at[g + b]], add=True)
        return carry

    lax.fori_loop(0, KCH // 2, lambda i, cy: outer(i * 2, cy), 0,
                  unroll=False)
    plsc.subcore_barrier()

    @pl.when(c == 0)
    def _():
        pltpu.sync_copy(acc_sh.at[pl.ds(s * ROWS_PT, ROWS_PT)],
                        out0_hbm.at[pl.ds(s * ROWS_PT, ROWS_PT)])

    @pl.when(c == 1)
    def _():
        pltpu.sync_copy(acc_sh.at[pl.ds(s * ROWS_PT, ROWS_PT)],
                        out1_hbm.at[pl.ds(s * ROWS_PT, ROWS_PT)])


_agg_kernel = pl.kernel(
    _agg_body,
    out_type=(jax.ShapeDtypeStruct((N_PAD, D_HID), jnp.float32),
              jax.ShapeDtypeStruct((N_PAD, D_HID), jnp.float32)),
    mesh=_MESH,
    scratch_types=[
        pltpu.VMEM((KCH, CHUNK), jnp.int32),
        pltpu.VMEM((KCH, CHUNK), jnp.int32),
        pltpu.VMEM((CHUNK, D_HID), jnp.float32),
        pltpu.VMEM((CHUNK, D_HID), jnp.float32),
        pltpu.VMEM_SHARED((N_PAD, D_HID), jnp.float32),
        pltpu.SemaphoreType.DMA,
        pltpu.SemaphoreType.DMA,
    ],
    compiler_params=_SC_PARAMS,
)


def _dinv(deg0_ref, deg1_ref):
    deg = deg0_ref[:, :1] + deg1_ref[:, :1] + 1.0
    return lax.rsqrt(deg)


def _stage_a_body(x_ref, w1_ref, deg0_ref, deg1_ref, h1p_ref):
    h = jnp.dot(x_ref[...], w1_ref[...], preferred_element_type=jnp.float32)
    h1p_ref[...] = _dinv(deg0_ref, deg1_ref) * h


def _stage_b_body(agg0_ref, agg1_ref, h1p_ref, deg0_ref, deg1_ref,
                  b1_ref, w2_ref, h2p_ref):
    dinv = _dinv(deg0_ref, deg1_ref)
    z = dinv * (agg0_ref[...] + agg1_ref[...] + h1p_ref[...]) + b1_ref[...]
    h2 = jnp.dot(jax.nn.relu(z), w2_ref[...],
                 preferred_element_type=jnp.float32)
    h2p_ref[...] = dinv * h2


def _stage_c_body(agg0_ref, agg1_ref, h2p_ref, deg0_ref, deg1_ref,
                  b2_ref, out_ref):
    dinv = _dinv(deg0_ref, deg1_ref)
    z = dinv * (agg0_ref[...] + agg1_ref[...] + h2p_ref[...]) + b2_ref[...]
    m = jnp.max(z, axis=1, keepdims=True)
    e = jnp.exp(z - m)
    out_ref[...] = (z - m) - jnp.log(jnp.sum(e, axis=1, keepdims=True))


def _tc_call(body, n_out):
    return pl.pallas_call(
        body,
        out_shape=[jax.ShapeDtypeStruct((N_PAD, D_HID), jnp.float32)
                   for _ in range(n_out)],
    )


@jax.jit
def kernel(x, edge_index, W1, b1, W2, b2):
    x_pad = jnp.pad(x, ((0, N_PAD - N_NODES), (0, 0)))
    src = edge_index[0].reshape(NW, KCH, CHUNK)
    dst = edge_index[1].reshape(NW, KCH, CHUNK)

    zeros16 = jnp.zeros((N_PAD, 16), jnp.float32)
    ones16 = jnp.ones((CHUNK, 16), jnp.float32)
    zeros64 = jnp.zeros((N_PAD, D_HID), jnp.float32)

    deg0, deg1 = _deg_kernel(dst, ones16, zeros16)

    (h1p,) = _tc_call(_stage_a_body, 1)(
        x_pad, W1, deg0, deg1)

    agg1_0, agg1_1 = _agg_kernel(h1p, src, dst, zeros64)

    (h2p,) = _tc_call(_stage_b_body, 1)(
        agg1_0, agg1_1, h1p, deg0, deg1,
        b1.reshape(1, D_HID), W2)

    agg2_0, agg2_1 = _agg_kernel(h2p, src, dst, zeros64)

    (out,) = _tc_call(_stage_c_body, 1)(
        agg2_0, agg2_1, h2p, deg0, deg1, b2.reshape(1, D_HID))

    return out[:N_NODES]


# speedup vs baseline: 41.9673x; 1.1674x over previous
import jax
import jax.numpy as jnp
from jax import lax
from jax.experimental import pallas as pl
from jax.experimental.pallas import tpu as pltpu
from jax.experimental.pallas import tpu_sc as plsc

N_NODES = 10000
D_IN = 128
D_HID = 64
E_EDGES = 320000

NC = 2
NS = 16
NW = NC * NS
N_PAD = 10240
ROWS_PT = N_PAD // NS
CHUNK = 128
KCH = 80
EPW = KCH * CHUNK
E_PAD = NW * EPW
NBUF = 4

_MESH = plsc.VectorSubcoreMesh(
    core_axis_name="c", subcore_axis_name="s", num_cores=NC, num_subcores=NS)

_SC_PARAMS = pltpu.CompilerParams(use_tc_tiling_on_sc=False)


def _wid():
    return lax.axis_index("s") * NC + lax.axis_index("c")


def _copy_out(acc_sh, c, s, out0_hbm, out1_hbm):
    @pl.when(c == 0)
    def _():
        pltpu.sync_copy(acc_sh.at[pl.ds(s * ROWS_PT, ROWS_PT)],
                        out0_hbm.at[pl.ds(s * ROWS_PT, ROWS_PT)])

    @pl.when(c == 1)
    def _():
        pltpu.sync_copy(acc_sh.at[pl.ds(s * ROWS_PT, ROWS_PT)],
                        out1_hbm.at[pl.ds(s * ROWS_PT, ROWS_PT)])


def _deg_body(dst_hbm, ones_hbm, zeros_hbm, out0_hbm, out1_hbm,
              idx_v, ones_v, acc_sh, sem):
    c = lax.axis_index("c")
    s = lax.axis_index("s")
    pltpu.sync_copy(zeros_hbm.at[pl.ds(s * ROWS_PT, ROWS_PT)],
                    acc_sh.at[pl.ds(s * ROWS_PT, ROWS_PT)])
    pltpu.sync_copy(ones_hbm, ones_v)
    pltpu.sync_copy(dst_hbm.at[_wid()], idx_v)
    plsc.subcore_barrier()

    @pl.loop(0, KCH)
    def _fire(g):
        pltpu.async_copy(ones_v, acc_sh.at[idx_v.at[g]], sem, add=True)

    @pl.loop(0, KCH)
    def _drain(g):
        pltpu.make_async_copy(ones_v, acc_sh.at[idx_v.at[g]], sem).wait()

    plsc.subcore_barrier()
    _copy_out(acc_sh, c, s, out0_hbm, out1_hbm)


_deg_kernel = pl.kernel(
    _deg_body,
    out_type=(jax.ShapeDtypeStruct((N_PAD, 16), jnp.float32),
              jax.ShapeDtypeStruct((N_PAD, 16), jnp.float32)),
    mesh=_MESH,
    scratch_types=[
        pltpu.VMEM((KCH, CHUNK), jnp.int32),
        pltpu.VMEM((CHUNK, 16), jnp.float32),
        pltpu.VMEM_SHARED((N_PAD, 16), jnp.float32),
        pltpu.SemaphoreType.DMA,
    ],
    compiler_params=_SC_PARAMS,
)


def _agg_body(h_hbm, src_hbm, dst_hbm, zeros_hbm, out0_hbm, out1_hbm,
              src_v, dst_v, b0, b1, b2, b3, acc_sh,
              g0, g1, g2, g3, s0, s1, s2, s3):
    c = lax.axis_index("c")
    s = lax.axis_index("s")
    bufs = (b0, b1, b2, b3)
    gsems = (g0, g1, g2, g3)
    ssems = (s0, s1, s2, s3)

    pltpu.sync_copy(zeros_hbm.at[pl.ds(s * ROWS_PT, ROWS_PT)],
                    acc_sh.at[pl.ds(s * ROWS_PT, ROWS_PT)])
    pltpu.sync_copy(src_hbm.at[_wid()], src_v)
    pltpu.sync_copy(dst_hbm.at[_wid()], dst_v)
    plsc.subcore_barrier()

    for j in range(NBUF - 1):
        pltpu.async_copy(h_hbm.at[src_v.at[j]], bufs[j], gsems[j])

    @pl.loop(0, KCH, step=NBUF)
    def _ring(k0):
        for b in range(NBUF):
            k = k0 + b
            pltpu.make_async_copy(h_hbm.at[src_v.at[k]],
                                  bufs[b], gsems[b]).wait()
            pltpu.async_copy(bufs[b], acc_sh.at[dst_v.at[k]],
                             ssems[b], add=True)
            nb = (b + NBUF - 1) % NBUF
            nxt = k + NBUF - 1

            @pl.when(nxt < KCH)
            def _():
                @pl.when(k >= 1)
                def _():
                    pltpu.make_async_copy(bufs[nb], acc_sh.at[dst_v.at[0]],
                                          ssems[nb]).wait()

                pltpu.async_copy(h_hbm.at[src_v.at[nxt]], bufs[nb], gsems[nb])

    for b in range(NBUF):
        pltpu.make_async_copy(bufs[b], acc_sh.at[dst_v.at[0]],
                              ssems[b]).wait()

    plsc.subcore_barrier()
    _copy_out(acc_sh, c, s, out0_hbm, out1_hbm)


_agg_kernel = pl.kernel(
    _agg_body,
    out_type=(jax.ShapeDtypeStruct((N_PAD, D_HID), jnp.float32),
              jax.ShapeDtypeStruct((N_PAD, D_HID), jnp.float32)),
    mesh=_MESH,
    scratch_types=(
        [pltpu.VMEM((KCH, CHUNK), jnp.int32)] * 2
        + [pltpu.VMEM((CHUNK, D_HID), jnp.float32)] * NBUF
        + [pltpu.VMEM_SHARED((N_PAD, D_HID), jnp.float32)]
        + [pltpu.SemaphoreType.DMA] * (2 * NBUF)
    ),
    compiler_params=_SC_PARAMS,
)


def _dinv(deg0_ref, deg1_ref, n):
    deg = deg0_ref[pl.ds(0, n), :1] + deg1_ref[pl.ds(0, n), :1] + 1.0
    return lax.rsqrt(deg)


def _stage_a_body(x_ref, w1_ref, deg0_ref, deg1_ref, h1p_ref):
    h = jnp.dot(x_ref[...], w1_ref[...], preferred_element_type=jnp.float32)
    h1p_ref[pl.ds(0, N_NODES), :] = _dinv(deg0_ref, deg1_ref, N_NODES) * h
    h1p_ref[pl.ds(N_NODES, N_PAD - N_NODES), :] = jnp.zeros(
        (N_PAD - N_NODES, D_HID), jnp.float32)


def _stage_b_body(agg0_ref, agg1_ref, h1p_ref, deg0_ref, deg1_ref,
                  b1_ref, w2_ref, h2p_ref):
    dinv = _dinv(deg0_ref, deg1_ref, N_PAD)
    z = dinv * (agg0_ref[...] + agg1_ref[...] + h1p_ref[...]) + b1_ref[...]
    h2 = jnp.dot(jax.nn.relu(z), w2_ref[...],
                 preferred_element_type=jnp.float32)
    h2p_ref[...] = dinv * h2


def _stage_c_body(agg0_ref, agg1_ref, h2p_ref, deg0_ref, deg1_ref,
                  b2_ref, out_ref):
    dinv = _dinv(deg0_ref, deg1_ref, N_NODES)
    n = pl.ds(0, N_NODES)
    z = dinv * (agg0_ref[n, :] + agg1_ref[n, :] + h2p_ref[n, :]) + b2_ref[...]
    m = jnp.max(z, axis=1, keepdims=True)
    e = jnp.exp(z - m)
    out_ref[...] = (z - m) - jnp.log(jnp.sum(e, axis=1, keepdims=True))


@jax.jit
def kernel(x, edge_index, W1, b1, W2, b2):
    n_extra = E_PAD - E_EDGES
    pad_src = jnp.arange(n_extra, dtype=jnp.int32) % N_PAD
    pad_dst = N_NODES + jnp.arange(n_extra, dtype=jnp.int32) % (N_PAD - N_NODES)
    src = jnp.concatenate([edge_index[0], pad_src]).reshape(NW, KCH, CHUNK)
    dst = jnp.concatenate([edge_index[1], pad_dst]).reshape(NW, KCH, CHUNK)

    zeros16 = jnp.zeros((N_PAD, 16), jnp.float32)
    ones16 = jnp.ones((CHUNK, 16), jnp.float32)
    zeros64 = jnp.zeros((N_PAD, D_HID), jnp.float32)

    deg0, deg1 = _deg_kernel(dst, ones16, zeros16)

    (h1p,) = pl.pallas_call(
        _stage_a_body,
        out_shape=[jax.ShapeDtypeStruct((N_PAD, D_HID), jnp.float32)],
    )(x, W1, deg0, deg1)

    agg1_0, agg1_1 = _agg_kernel(h1p, src, dst, zeros64)

    (h2p,) = pl.pallas_call(
        _stage_b_body,
        out_shape=[jax.ShapeDtypeStruct((N_PAD, D_HID), jnp.float32)],
    )(agg1_0, agg1_1, h1p, deg0, deg1, b1.reshape(1, D_HID), W2)

    agg2_0, agg2_1 = _agg_kernel(h2p, src, dst, zeros64)

    (out,) = pl.pallas_call(
        _stage_c_body,
        out_shape=[jax.ShapeDtypeStruct((N_NODES, D_HID), jnp.float32)],
    )(agg2_0, agg2_1, h2p, deg0, deg1, b2.reshape(1, D_HID))

    return out


# speedup vs baseline: 44.6378x; 1.0636x over previous
import jax
import jax.numpy as jnp
from jax import lax
from jax.experimental import pallas as pl
from jax.experimental.pallas import tpu as pltpu
from jax.experimental.pallas import tpu_sc as plsc

N_NODES = 10000
D_IN = 128
D_HID = 64
E_EDGES = 320000

NC = 2
NS = 16
NW = NC * NS
N_PAD = 10240
ROWS_PT = N_PAD // NS
CHUNK = 128
KCH = 80
EPW = KCH * CHUNK
E_PAD = NW * EPW
NBUF = 4

_MESH = plsc.VectorSubcoreMesh(
    core_axis_name="c", subcore_axis_name="s", num_cores=NC, num_subcores=NS)

_SC_PARAMS = pltpu.CompilerParams(use_tc_tiling_on_sc=False)


def _wid():
    return lax.axis_index("s") * NC + lax.axis_index("c")


def _fill(ref, value):
    rows, cols = ref.shape

    @pl.loop(0, rows)
    def _(i):
        for j in range(cols // 16):
            ref[i, pl.ds(j * 16, 16)] = jnp.full((16,), value, jnp.float32)


def _zero_acc(acc_sh, s, zbuf):
    _fill(zbuf, 0.0)
    reps = ROWS_PT // zbuf.shape[0]
    for r in range(reps):
        pltpu.sync_copy(
            zbuf, acc_sh.at[pl.ds(s * ROWS_PT + r * zbuf.shape[0],
                                  zbuf.shape[0])])


def _copy_out(acc_sh, c, s, out0_hbm, out1_hbm):
    @pl.when(c == 0)
    def _():
        pltpu.sync_copy(acc_sh.at[pl.ds(s * ROWS_PT, ROWS_PT)],
                        out0_hbm.at[pl.ds(s * ROWS_PT, ROWS_PT)])

    @pl.when(c == 1)
    def _():
        pltpu.sync_copy(acc_sh.at[pl.ds(s * ROWS_PT, ROWS_PT)],
                        out1_hbm.at[pl.ds(s * ROWS_PT, ROWS_PT)])


def _deg_body(edges_hbm, out0_hbm, out1_hbm, idx_v, ones_v, acc_sh, sem):
    c = lax.axis_index("c")
    s = lax.axis_index("s")
    _zero_acc(acc_sh, s, ones_v)
    _fill(ones_v, 1.0)
    pltpu.sync_copy(edges_hbm.at[1, _wid()], idx_v)
    plsc.subcore_barrier()

    @pl.loop(0, KCH)
    def _fire(g):
        pltpu.async_copy(ones_v, acc_sh.at[idx_v.at[g]], sem, add=True)

    @pl.loop(0, KCH)
    def _drain(g):
        pltpu.make_async_copy(ones_v, acc_sh.at[idx_v.at[g]], sem).wait()

    plsc.subcore_barrier()
    _copy_out(acc_sh, c, s, out0_hbm, out1_hbm)


_deg_kernel = pl.kernel(
    _deg_body,
    out_type=(jax.ShapeDtypeStruct((N_PAD, 16), jnp.float32),
              jax.ShapeDtypeStruct((N_PAD, 16), jnp.float32)),
    mesh=_MESH,
    scratch_types=[
        pltpu.VMEM((KCH, CHUNK), jnp.int32),
        pltpu.VMEM((CHUNK, 16), jnp.float32),
        pltpu.VMEM_SHARED((N_PAD, 16), jnp.float32),
        pltpu.SemaphoreType.DMA,
    ],
    compiler_params=_SC_PARAMS,
)


def _agg_body(h_hbm, edges_hbm, out0_hbm, out1_hbm,
              src_v, dst_v, b0, b1, b2, b3, acc_sh,
              g0, g1, g2, g3, s0, s1, s2, s3):
    c = lax.axis_index("c")
    s = lax.axis_index("s")
    bufs = (b0, b1, b2, b3)
    gsems = (g0, g1, g2, g3)
    ssems = (s0, s1, s2, s3)

    _zero_acc(acc_sh, s, b0)
    pltpu.sync_copy(edges_hbm.at[0, _wid()], src_v)
    pltpu.sync_copy(edges_hbm.at[1, _wid()], dst_v)
    plsc.subcore_barrier()

    for j in range(NBUF - 1):
        pltpu.async_copy(h_hbm.at[src_v.at[j]], bufs[j], gsems[j])

    @pl.loop(0, KCH, step=NBUF)
    def _ring(k0):
        for b in range(NBUF):
            k = k0 + b
            pltpu.make_async_copy(h_hbm.at[src_v.at[k]],
                                  bufs[b], gsems[b]).wait()
            pltpu.async_copy(bufs[b], acc_sh.at[dst_v.at[k]],
                             ssems[b], add=True)
            nb = (b + NBUF - 1) % NBUF
            nxt = k + NBUF - 1

            @pl.when(nxt < KCH)
            def _():
                @pl.when(k >= 1)
                def _():
                    pltpu.make_async_copy(bufs[nb], acc_sh.at[dst_v.at[0]],
                                          ssems[nb]).wait()

                pltpu.async_copy(h_hbm.at[src_v.at[nxt]], bufs[nb], gsems[nb])

    for b in range(NBUF):
        pltpu.make_async_copy(bufs[b], acc_sh.at[dst_v.at[0]],
                              ssems[b]).wait()

    plsc.subcore_barrier()
    _copy_out(acc_sh, c, s, out0_hbm, out1_hbm)


_agg_kernel = pl.kernel(
    _agg_body,
    out_type=(jax.ShapeDtypeStruct((N_PAD, D_HID), jnp.float32),
              jax.ShapeDtypeStruct((N_PAD, D_HID), jnp.float32)),
    mesh=_MESH,
    scratch_types=(
        [pltpu.VMEM((KCH, CHUNK), jnp.int32)] * 2
        + [pltpu.VMEM((CHUNK, D_HID), jnp.float32)] * NBUF
        + [pltpu.VMEM_SHARED((N_PAD, D_HID), jnp.float32)]
        + [pltpu.SemaphoreType.DMA] * (2 * NBUF)
    ),
    compiler_params=_SC_PARAMS,
)


def _dinv(deg0_ref, deg1_ref, n):
    deg = deg0_ref[pl.ds(0, n), :1] + deg1_ref[pl.ds(0, n), :1] + 1.0
    return lax.rsqrt(deg)


def _stage_a_body(x_ref, w1_ref, deg0_ref, deg1_ref, h1p_ref):
    h = jnp.dot(x_ref[...], w1_ref[...], preferred_element_type=jnp.float32)
    h1p_ref[pl.ds(0, N_NODES), :] = _dinv(deg0_ref, deg1_ref, N_NODES) * h
    h1p_ref[pl.ds(N_NODES, N_PAD - N_NODES), :] = jnp.zeros(
        (N_PAD - N_NODES, D_HID), jnp.float32)


def _stage_b_body(agg0_ref, agg1_ref, h1p_ref, deg0_ref, deg1_ref,
                  b1_ref, w2_ref, h2p_ref):
    dinv = _dinv(deg0_ref, deg1_ref, N_PAD)
    z = dinv * (agg0_ref[...] + agg1_ref[...] + h1p_ref[...]) + b1_ref[...]
    h2 = jnp.dot(jax.nn.relu(z), w2_ref[...],
                 preferred_element_type=jnp.float32)
    h2p_ref[...] = dinv * h2


def _stage_c_body(agg0_ref, agg1_ref, h2p_ref, deg0_ref, deg1_ref,
                  b2_ref, out_ref):
    dinv = _dinv(deg0_ref, deg1_ref, N_NODES)
    n = pl.ds(0, N_NODES)
    z = dinv * (agg0_ref[n, :] + agg1_ref[n, :] + h2p_ref[n, :]) + b2_ref[...]
    m = jnp.max(z, axis=1, keepdims=True)
    e = jnp.exp(z - m)
    out_ref[...] = (z - m) - jnp.log(jnp.sum(e, axis=1, keepdims=True))


@jax.jit
def kernel(x, edge_index, W1, b1, W2, b2):
    n_extra = E_PAD - E_EDGES
    pad_src = jnp.arange(n_extra, dtype=jnp.int32) % N_PAD
    pad_dst = N_NODES + jnp.arange(n_extra, dtype=jnp.int32) % (N_PAD - N_NODES)
    pads = jnp.stack([pad_src, pad_dst])
    edges = jnp.concatenate([edge_index, pads], axis=1).reshape(
        2, NW, KCH, CHUNK)

    deg0, deg1 = _deg_kernel(edges)

    (h1p,) = pl.pallas_call(
        _stage_a_body,
        out_shape=[jax.ShapeDtypeStruct((N_PAD, D_HID), jnp.float32)],
    )(x, W1, deg0, deg1)

    agg1_0, agg1_1 = _agg_kernel(h1p, edges)

    (h2p,) = pl.pallas_call(
        _stage_b_body,
        out_shape=[jax.ShapeDtypeStruct((N_PAD, D_HID), jnp.float32)],
    )(agg1_0, agg1_1, h1p, deg0, deg1, b1.reshape(1, D_HID), W2)

    agg2_0, agg2_1 = _agg_kernel(h2p, edges)

    (out,) = pl.pallas_call(
        _stage_c_body,
        out_shape=[jax.ShapeDtypeStruct((N_NODES, D_HID), jnp.float32)],
    )(agg2_0, agg2_1, h2p, deg0, deg1, b2.reshape(1, D_HID))

    return out


# speedup vs baseline: 48.9925x; 1.0976x over previous
import jax
import jax.numpy as jnp
from jax import lax
from jax.experimental import pallas as pl
from jax.experimental.pallas import tpu as pltpu
from jax.experimental.pallas import tpu_sc as plsc

N_NODES = 10000
D_IN = 128
D_HID = 64
E_EDGES = 320000

NC = 2
NS = 16
NW = NC * NS
N_PAD = 10240
ROWS_PT = N_PAD // NS
CHUNK = 128
KCH = 80
EPW = KCH * CHUNK
E_PAD = NW * EPW
NBUF = 4

NH = N_NODES // 2
NPH = N_PAD // 2

_MESH = plsc.VectorSubcoreMesh(
    core_axis_name="c", subcore_axis_name="s", num_cores=NC, num_subcores=NS)

_SC_PARAMS = pltpu.CompilerParams(use_tc_tiling_on_sc=False)


def _wid():
    return lax.axis_index("s") * NC + lax.axis_index("c")


def _fill(ref, value):
    rows, cols = ref.shape

    @pl.loop(0, rows)
    def _(i):
        for j in range(cols // 16):
            ref[i, pl.ds(j * 16, 16)] = jnp.full((16,), value, jnp.float32)


def _zero_acc(acc_sh, s, zbuf):
    _fill(zbuf, 0.0)
    reps = ROWS_PT // zbuf.shape[0]
    for r in range(reps):
        pltpu.sync_copy(
            zbuf, acc_sh.at[pl.ds(s * ROWS_PT + r * zbuf.shape[0],
                                  zbuf.shape[0])])


def _copy_out(acc_sh, c, s, out0_hbm, out1_hbm):
    @pl.when(c == 0)
    def _():
        pltpu.sync_copy(acc_sh.at[pl.ds(s * ROWS_PT, ROWS_PT)],
                        out0_hbm.at[pl.ds(s * ROWS_PT, ROWS_PT)])

    @pl.when(c == 1)
    def _():
        pltpu.sync_copy(acc_sh.at[pl.ds(s * ROWS_PT, ROWS_PT)],
                        out1_hbm.at[pl.ds(s * ROWS_PT, ROWS_PT)])


def _deg_body(edges_hbm, out0_hbm, out1_hbm, idx_v, ones_v, acc_sh, sem):
    c = lax.axis_index("c")
    s = lax.axis_index("s")
    _zero_acc(acc_sh, s, ones_v)
    _fill(ones_v, 1.0)
    pltpu.sync_copy(edges_hbm.at[1, _wid()], idx_v)
    plsc.subcore_barrier()

    @pl.loop(0, KCH)
    def _fire(g):
        pltpu.async_copy(ones_v, acc_sh.at[idx_v.at[g]], sem, add=True)

    @pl.loop(0, KCH)
    def _drain(g):
        pltpu.make_async_copy(ones_v, acc_sh.at[idx_v.at[g]], sem).wait()

    plsc.subcore_barrier()
    _copy_out(acc_sh, c, s, out0_hbm, out1_hbm)


_deg_kernel = pl.kernel(
    _deg_body,
    out_type=(jax.ShapeDtypeStruct((N_PAD, D_HID), jnp.float32),
              jax.ShapeDtypeStruct((N_PAD, D_HID), jnp.float32)),
    mesh=_MESH,
    scratch_types=[
        pltpu.VMEM((KCH, CHUNK), jnp.int32),
        pltpu.VMEM((CHUNK, D_HID), jnp.float32),
        pltpu.VMEM_SHARED((N_PAD, D_HID), jnp.float32),
        pltpu.SemaphoreType.DMA,
    ],
    compiler_params=_SC_PARAMS,
)


def _agg_body(h_hbm, edges_hbm, out0_hbm, out1_hbm,
              src_v, dst_v, b0, b1, b2, b3, acc_sh,
              g0, g1, g2, g3, s0, s1, s2, s3):
    c = lax.axis_index("c")
    s = lax.axis_index("s")
    bufs = (b0, b1, b2, b3)
    gsems = (g0, g1, g2, g3)
    ssems = (s0, s1, s2, s3)

    _zero_acc(acc_sh, s, b0)
    pltpu.sync_copy(edges_hbm.at[0, _wid()], src_v)
    pltpu.sync_copy(edges_hbm.at[1, _wid()], dst_v)
    plsc.subcore_barrier()

    for j in range(NBUF - 1):
        pltpu.async_copy(h_hbm.at[src_v.at[j]], bufs[j], gsems[j])

    @pl.loop(0, KCH, step=NBUF)
    def _ring(k0):
        for b in range(NBUF):
            k = k0 + b
            pltpu.make_async_copy(h_hbm.at[src_v.at[k]],
                                  bufs[b], gsems[b]).wait()
            pltpu.async_copy(bufs[b], acc_sh.at[dst_v.at[k]],
                             ssems[b], add=True)
            nb = (b + NBUF - 1) % NBUF
            nxt = k + NBUF - 1

            @pl.when(nxt < KCH)
            def _():
                @pl.when(k >= 1)
                def _():
                    pltpu.make_async_copy(bufs[nb], acc_sh.at[dst_v.at[0]],
                                          ssems[nb]).wait()

                pltpu.async_copy(h_hbm.at[src_v.at[nxt]], bufs[nb], gsems[nb])

    for b in range(NBUF):
        pltpu.make_async_copy(bufs[b], acc_sh.at[dst_v.at[0]],
                              ssems[b]).wait()

    plsc.subcore_barrier()
    _copy_out(acc_sh, c, s, out0_hbm, out1_hbm)


_agg_kernel = pl.kernel(
    _agg_body,
    out_type=(jax.ShapeDtypeStruct((N_PAD, D_HID), jnp.float32),
              jax.ShapeDtypeStruct((N_PAD, D_HID), jnp.float32)),
    mesh=_MESH,
    scratch_types=(
        [pltpu.VMEM((KCH, CHUNK), jnp.int32)] * 2
        + [pltpu.VMEM((CHUNK, D_HID), jnp.float32)] * NBUF
        + [pltpu.VMEM_SHARED((N_PAD, D_HID), jnp.float32)]
        + [pltpu.SemaphoreType.DMA] * (2 * NBUF)
    ),
    compiler_params=_SC_PARAMS,
)


def _pair_matmul(zp, w):
    k = w.shape[0]
    left = jnp.dot(zp[:, 0:k], w, preferred_element_type=jnp.float32)
    right = jnp.dot(zp[:, k:2 * k], w, preferred_element_type=jnp.float32)
    return jnp.concatenate([left, right], axis=1)


def _stage_a_body(xp_ref, w1_ref, s0_ref, s1_ref, h1p_ref, dinv_ref):
    dinv = lax.rsqrt(s0_ref[...] + s1_ref[...] + 1.0)
    dinv_ref[...] = dinv
    h = _pair_matmul(xp_ref[...], w1_ref[...])
    h1p_ref[pl.ds(0, NH), :] = dinv[0:NH, :] * h
    h1p_ref[pl.ds(NH, NPH - NH), :] = jnp.zeros((NPH - NH, 2 * D_HID),
                                                jnp.float32)


def _stage_b_body(agg0_ref, agg1_ref, h1p_ref, dinv_ref, b1_ref, w2_ref,
                  h2p_ref):
    dinv = dinv_ref[...]
    z = dinv * (agg0_ref[...] + agg1_ref[...] + h1p_ref[...]) + b1_ref[...]
    h2p_ref[...] = dinv * _pair_matmul(jax.nn.relu(z), w2_ref[...])


def _stage_c_body(agg0_ref, agg1_ref, h2p_ref, dinv_ref, b2_ref, out_ref):
    n = pl.ds(0, NH)
    z = (dinv_ref[n, :] * (agg0_ref[n, :] + agg1_ref[n, :] + h2p_ref[n, :])
         + b2_ref[...])
    for half in range(2):
        zh = z[:, half * D_HID:(half + 1) * D_HID]
        m = jnp.max(zh, axis=1, keepdims=True)
        e = jnp.exp(zh - m)
        out_ref[:, pl.ds(half * D_HID, D_HID)] = (
            (zh - m) - jnp.log(jnp.sum(e, axis=1, keepdims=True)))


@jax.jit
def kernel(x, edge_index, W1, b1, W2, b2):
    n_extra = E_PAD - E_EDGES
    pad_src = jnp.arange(n_extra, dtype=jnp.int32) % N_PAD
    pad_dst = N_NODES + jnp.arange(n_extra, dtype=jnp.int32) % (N_PAD - N_NODES)
    pads = jnp.stack([pad_src, pad_dst])
    edges = jnp.concatenate([edge_index, pads], axis=1).reshape(
        2, NW, KCH, CHUNK)

    xp = x.reshape(NH, 2 * D_IN)
    b1p = jnp.tile(b1, 2).reshape(1, 2 * D_HID)
    b2p = jnp.tile(b2, 2).reshape(1, 2 * D_HID)

    deg0, deg1 = _deg_kernel(edges)

    h1p, dinvp = pl.pallas_call(
        _stage_a_body,
        out_shape=[jax.ShapeDtypeStruct((NPH, 2 * D_HID), jnp.float32),
                   jax.ShapeDtypeStruct((NPH, 2 * D_HID), jnp.float32)],
    )(xp, W1, deg0.reshape(NPH, 2 * D_HID), deg1.reshape(NPH, 2 * D_HID))

    agg1_0, agg1_1 = _agg_kernel(h1p.reshape(N_PAD, D_HID), edges)

    (h2p,) = pl.pallas_call(
        _stage_b_body,
        out_shape=[jax.ShapeDtypeStruct((NPH, 2 * D_HID), jnp.float32)],
    )(agg1_0.reshape(NPH, 2 * D_HID), agg1_1.reshape(NPH, 2 * D_HID),
      h1p, dinvp, b1p, W2)

    agg2_0, agg2_1 = _agg_kernel(h2p.reshape(N_PAD, D_HID), edges)

    (outp,) = pl.pallas_call(
        _stage_c_body,
        out_shape=[jax.ShapeDtypeStruct((NH, 2 * D_HID), jnp.float32)],
    )(agg2_0.reshape(NPH, 2 * D_HID), agg2_1.reshape(NPH, 2 * D_HID),
      h2p, dinvp, b2p)

    return outp.reshape(N_NODES, D_HID)
